# baseline (speedup 1.0000x reference)
import jax
import jax.numpy as jnp
from jax import lax
from jax.experimental import pallas as pl
from jax.experimental.pallas import tpu as pltpu

_B, _D, _K = 128, 32768, 64
_DT = 512
_ND = _D // _DT
_RB = 8
_NR = _B // _RB
_W = 128
_NBW = _DT // _W
_MAXR = 31

_INTERPRET = False


def _acc_kernel(x_ref, u_ref, v_ref):
    @pl.when(pl.program_id(0) == 0)
    def _():
        v_ref[...] = jnp.zeros_like(v_ref)
    v_ref[...] += jnp.dot(x_ref[...], u_ref[...],
                          preferred_element_type=jnp.float32)


def _score_kernel(x_ref, g_ref, u_ref, th_ref, v_ref,
                  outx_ref, pert_ref, bm_ref, m_ref, se_ref,
                  macc, seacc):
    j = pl.program_id(0)
    x = x_ref[...]
    grad = th_ref[...] + lax.dot_general(
        v_ref[...], u_ref[...], (((1,), (1,)), ((), ())),
        preferred_element_type=jnp.float32)
    s = (0.5 - x) * grad
    pert = s + g_ref[...]
    outx_ref[...] = x
    pert_ref[...] = pert
    bm_ref[...] = jnp.max(pert.reshape(_RB, _NBW, _W), axis=-1)[None]
    tm = jnp.max(s, axis=-1, keepdims=True)
    tse = jnp.sum(jnp.exp(s - tm), axis=-1, keepdims=True)
    row0 = pl.program_id(1) * _RB

    @pl.when(j == 0)
    def _():
        macc[pl.ds(row0, _RB), :] = tm
        seacc[pl.ds(row0, _RB), :] = tse

    @pl.when(j > 0)
    def _():
        m_old = macc[pl.ds(row0, _RB), :]
        se_old = seacc[pl.ds(row0, _RB), :]
        m_new = jnp.maximum(m_old, tm)
        macc[pl.ds(row0, _RB), :] = m_new
        seacc[pl.ds(row0, _RB), :] = (se_old * jnp.exp(m_old - m_new)
                                      + tse * jnp.exp(tm - m_new))

    @pl.when(j == _ND - 1)
    def _():
        m_ref[...] = macc[pl.ds(row0, _RB), :]
        se_ref[...] = seacc[pl.ds(row0, _RB), :]


def kernel(x, theta, U, radius, gumbel, u_accept):
    f32 = jnp.float32
    x = x.astype(f32)
    th2 = theta.reshape(1, _D)

    v = pl.pallas_call(
        _acc_kernel,
        grid=(_ND,),
        in_specs=[
            pl.BlockSpec((_B, _DT), lambda j: (0, j)),
            pl.BlockSpec((_DT, _K), lambda j: (j, 0)),
        ],
        out_specs=pl.BlockSpec((_B, _K), lambda j: (0, 0)),
        out_shape=jax.ShapeDtypeStruct((_B, _K), f32),
        interpret=_INTERPRET,
    )(x, U)

    outs = pl.pallas_call(
        _score_kernel,
        grid=(_ND, _NR),
        in_specs=[
            pl.BlockSpec((_RB, _DT), lambda j, i: (i, j)),
            pl.BlockSpec((_RB, _DT), lambda j, i: (i, j)),
            pl.BlockSpec((_DT, _K), lambda j, i: (j, 0)),
            pl.BlockSpec((1, _DT), lambda j, i: (0, j)),
            pl.BlockSpec((_RB, _K), lambda j, i: (i, 0)),
        ],
        out_specs=[
            pl.BlockSpec((_RB, _DT), lambda j, i: (i, j)),
            pl.BlockSpec((_RB, _DT), lambda j, i: (i, j)),
            pl.BlockSpec((1, _RB, _NBW), lambda j, i: (j, i, 0)),
            pl.BlockSpec((_RB, 1), lambda j, i: (i, 0)),
            pl.BlockSpec((_RB, 1), lambda j, i: (i, 0)),
        ],
        out_shape=[
            jax.ShapeDtypeStruct((_B, _D), f32),
            jax.ShapeDtypeStruct((_B, _D), f32),
            jax.ShapeDtypeStruct((_ND, _B, _NBW), f32),
            jax.ShapeDtypeStruct((_B, 1), f32),
            jax.ShapeDtypeStruct((_B, 1), f32),
        ],
        scratch_shapes=[
            pltpu.VMEM((_B, 1), f32),
            pltpu.VMEM((_B, 1), f32),
        ],
        interpret=_INTERPRET,
    )(x, gumbel, U, th2, v)
    out_x, pert, bm, m, se = outs

    vals, idx = lax.top_k(pert, _MAXR)
    g_sel = jnp.take_along_axis(gumbel, idx, axis=1)
    s_sel = vals - g_sel
    x_sel = jnp.take_along_axis(x, idx, axis=1)
    delta = 1.0 - 2.0 * x_sel
    flip = (jnp.arange(_MAXR)[None, :] < radius).astype(f32)
    th_sel = theta[idx]
    u_sel = U[idx]
    w = jnp.einsum('bj,bjk->bk', flip * delta, u_sel)
    dscore = (jnp.sum(flip * delta * th_sel, axis=-1)
              + jnp.sum(v * w, axis=-1) + 0.5 * jnp.sum(w * w, axis=-1))
    sum_s = jnp.sum(flip * s_sel, axis=-1)
    a = dscore - 2.0 * sum_s
    se_y = se[:, 0] + jnp.sum(
        flip * (jnp.exp(-s_sel - m) - jnp.exp(s_sel - m)), axis=-1)
    accept = (jnp.exp(jnp.clip(a, -60.0, 60.0)) * se[:, 0]
              > u_accept * jnp.maximum(se_y, 1e-30))
    newbits = jnp.where((accept[:, None] & (flip > 0)), 1.0 - x_sel, x_sel)
    rows = jnp.arange(_B)[:, None]
    return out_x.at[rows, idx].set(newbits)


# speedup vs baseline: 1.5065x; 1.5065x over previous
import functools

import jax
import jax.numpy as jnp
from jax import lax
from jax.experimental import pallas as pl
from jax.experimental.pallas import tpu as pltpu
from jax.experimental.pallas import tpu_sc as plsc

_B, _D, _K = 128, 32768, 64
_DT = 512
_ND = _D // _DT
_RB = 8
_NR = _B // _RB
_W = 128
_NBW = _DT // _W
_NBLK = _D // _W
_MAXR = 31
_NW = 32
_RPW = _B // _NW
_POOL = _MAXR * _W + 16
_NEG = -3.0e38

_INTERPRET = False


def _acc_kernel(x_ref, u_ref, v_ref):
    @pl.when(pl.program_id(0) == 0)
    def _():
        v_ref[...] = jnp.zeros_like(v_ref)
    v_ref[...] += jnp.dot(x_ref[...], u_ref[...],
                          preferred_element_type=jnp.float32)


def _score_kernel(x_ref, g_ref, u_ref, th_ref, v_ref,
                  pert_ref, bm_ref, m_ref, se_ref,
                  macc, seacc):
    j = pl.program_id(0)
    x = x_ref[...]
    grad = th_ref[...] + lax.dot_general(
        v_ref[...], u_ref[...], (((1,), (1,)), ((), ())),
        preferred_element_type=jnp.float32)
    s = (0.5 - x) * grad
    pert = s + g_ref[...]
    pert_ref[...] = pert
    bm_ref[...] = jnp.max(pert.reshape(_RB, _NBW, _W), axis=-1)[None]
    tm = jnp.max(s, axis=-1, keepdims=True)
    tse = jnp.sum(jnp.exp(s - tm), axis=-1, keepdims=True)
    row0 = pl.program_id(1) * _RB

    @pl.when(j == 0)
    def _():
        macc[pl.ds(row0, _RB), :] = tm
        seacc[pl.ds(row0, _RB), :] = tse

    @pl.when(j > 0)
    def _():
        m_old = macc[pl.ds(row0, _RB), :]
        se_old = seacc[pl.ds(row0, _RB), :]
        m_new = jnp.maximum(m_old, tm)
        macc[pl.ds(row0, _RB), :] = m_new
        seacc[pl.ds(row0, _RB), :] = (se_old * jnp.exp(m_old - m_new)
                                      + tse * jnp.exp(tm - m_new))

    @pl.when(j == _ND - 1)
    def _():
        m_ref[...] = macc[pl.ds(row0, _RB), :]
        se_ref[...] = seacc[pl.ds(row0, _RB), :]


def _sc_body(x_hbm, pblk_hbm, bm_hbm, m_hbm, se_hbm, rad_hbm, uac_hbm,
             v_hbm, th_hbm, utab_hbm, gf_hbm,
             out_hbm,
             xrow_v, bm_v, blkid_v, cbase_v, blocks_v, pool_v, poolix_v,
             topix_v, topval_v, gidx_v, uidx_v, gsel_v, thsel_v, usel_v,
             vrow_v, fd_v, rad_v, uac_v, m_v, se_v, sem, sem2):
    i32 = jnp.int32
    f32 = jnp.float32
    lanes = lax.iota(i32, 16)
    zi = jnp.zeros((16,), i32)
    zf = jnp.zeros((16,), f32)

    wid = lax.axis_index("s") * 2 + lax.axis_index("c")
    base = wid * _RPW

    pltpu.sync_copy(rad_hbm, rad_v)
    pltpu.sync_copy(uac_hbm, uac_v)
    pltpu.sync_copy(m_hbm, m_v)
    pltpu.sync_copy(se_hbm, se_v)

    def row_body(rr, carry):
        r = base + rr
        cpx = pltpu.async_copy(x_hbm.at[r], xrow_v, sem2)
        pltpu.sync_copy(bm_hbm.at[r], bm_v)
        pltpu.sync_copy(v_hbm.at[r], vrow_v)

        blkid_v[pl.ds(0, 16)] = zi
        blkid_v[pl.ds(16, 16)] = zi
        cbase_v[pl.ds(0, 16)] = zi
        cbase_v[pl.ds(16, 16)] = zi

        def ext_blk(j, _t):
            rv = jnp.full((16,), _NEG, f32)
            for k in range(16):
                rv = jnp.maximum(rv, bm_v[pl.ds(16 * k, 16)])
            gmax = jnp.max(rv)
            posv = jnp.full((16,), -1, i32)
            for k in range(16):
                pv = bm_v[pl.ds(16 * k, 16)]
                posv = jnp.maximum(
                    posv, jnp.where(pv == gmax, lanes + 16 * k, -1))
            pos = jnp.max(posv)
            plsc.store_scatter(bm_v, [zi + pos], zf + _NEG,
                               mask=lanes == 0)
            plsc.store_scatter(blkid_v, [zi + j], zi + (r * _NBLK + pos),
                               mask=lanes == 0)
            plsc.store_scatter(cbase_v, [zi + j], zi + pos * _W,
                               mask=lanes == 0)
            return gmax
        t31 = lax.fori_loop(0, _MAXR, ext_blk, _NEG)

        pltpu.async_copy(pblk_hbm.at[blkid_v], blocks_v, sem).wait()

        t31v = zf + t31

        def comp_blk(jj, cur):
            cb = plsc.load_gather(cbase_v, [zi + jj])
            for k in range(8):
                pv = blocks_v[jj, pl.ds(16 * k, 16)]
                msk = pv >= t31v
                plsc.store_compressed(pool_v.at[pl.ds(cur, 16)], pv, mask=msk)
                plsc.store_compressed(poolix_v.at[pl.ds(cur, 16)],
                                      cb + (16 * k) + lanes, mask=msk)
                cur = cur + jnp.max(
                    plsc.all_reduce_population_count(msk))
            return cur
        psize = lax.fori_loop(0, _MAXR, comp_blk, jnp.int32(0))
        pool_v[pl.ds(psize, 16)] = zf + _NEG
        poolix_v[pl.ds(psize, 16)] = zi

        nv = (psize + 15) // 16
        topix_v[pl.ds(0, 16)] = zi
        topix_v[pl.ds(16, 16)] = zi

        def ext_pool(j, _u):
            def scan1(k, rv):
                return jnp.maximum(rv, pool_v[pl.ds(16 * k, 16)])
            rv = lax.fori_loop(0, nv, scan1, jnp.full((16,), _NEG, f32))
            gmax = jnp.max(rv)

            def scan2(k, c):
                posv, idxv = c
                pv = pool_v[pl.ds(16 * k, 16)]
                hit = pv == gmax
                posv = jnp.maximum(
                    posv, jnp.where(hit, lanes + 16 * k, -1))
                idxv = jnp.maximum(
                    idxv, jnp.where(hit, poolix_v[pl.ds(16 * k, 16)], -1))
                return posv, idxv
            posv, idxv = lax.fori_loop(
                0, nv, scan2,
                (jnp.full((16,), -1, i32), jnp.full((16,), -1, i32)))
            pos = jnp.max(posv)
            col = jnp.max(idxv)
            plsc.store_scatter(pool_v, [zi + pos], zf + _NEG,
                               mask=lanes == 0)
            plsc.store_scatter(topix_v, [zi + j], zi + col,
                               mask=lanes == 0)
            plsc.store_scatter(topval_v, [zi + j], zf + gmax,
                               mask=lanes == 0)
            return 0
        lax.fori_loop(0, _MAXR, ext_pool, 0)

        tix0 = topix_v[pl.ds(0, 16)]
        tix1 = topix_v[pl.ds(16, 16)]
        gidx_v[pl.ds(0, 16)] = tix0 + r * _D
        gidx_v[pl.ds(16, 16)] = tix1 + r * _D
        uidx_v[pl.ds(0, 16)] = tix0 // 2
        uidx_v[pl.ds(16, 16)] = tix1 // 2
        pltpu.async_copy(gf_hbm.at[gidx_v], gsel_v, sem).wait()
        pltpu.async_copy(th_hbm.at[topix_v], thsel_v, sem).wait()
        pltpu.async_copy(utab_hbm.at[uidx_v], usel_v, sem).wait()
        cpx.wait()
        x0 = plsc.load_gather(xrow_v, [tix0])
        x1 = plsc.load_gather(xrow_v, [tix1])

        rad = plsc.load_gather(rad_v, [zi + r])
        uacc = plsc.load_gather(uac_v, [zi + r])
        mrow = plsc.load_gather(m_v, [zi + r])
        serow = plsc.load_gather(se_v, [zi + r])

        tv0 = topval_v[pl.ds(0, 16)]
        tv1 = topval_v[pl.ds(16, 16)]
        s0 = tv0 - gsel_v[pl.ds(0, 16)]
        s1 = tv1 - gsel_v[pl.ds(16, 16)]
        d0 = 1.0 - 2.0 * x0
        d1 = 1.0 - 2.0 * x1
        f0 = jnp.where(lanes < rad, 1.0, 0.0)
        f1 = jnp.where(lanes + 16 < rad, 1.0, 0.0)
        fd0 = f0 * d0
        fd1 = f1 * d1
        fd_v[pl.ds(0, 16)] = fd0
        fd_v[pl.ds(16, 16)] = fd1

        def wacc_body(j, wc):
            fdj = plsc.load_gather(fd_v, [zi + j])
            tj = jnp.max(plsc.load_gather(topix_v, [zi + j]))
            half = (tj % 2) * 64
            return tuple(
                wc[c] + fdj * usel_v[j, pl.ds(half + 16 * c, 16)]
                for c in range(4))
        w = lax.fori_loop(0, _MAXR, wacc_body, (zf, zf, zf, zf))

        dth = jnp.sum(fd0 * thsel_v[pl.ds(0, 16)]
                      + fd1 * thsel_v[pl.ds(16, 16)])
        vw = jnp.float32(0.0)
        ww = jnp.float32(0.0)
        for c in range(4):
            vv = vrow_v[pl.ds(16 * c, 16)]
            vw = vw + jnp.sum(vv * w[c])
            ww = ww + jnp.sum(w[c] * w[c])
        dscore = dth + vw + 0.5 * ww
        sum_s = jnp.sum(f0 * s0 + f1 * s1)
        a = dscore - 2.0 * sum_s

        corr = (f0 * (jnp.exp(-s0 - mrow) - jnp.exp(s0 - mrow))
                + f1 * (jnp.exp(-s1 - mrow) - jnp.exp(s1 - mrow)))
        se_y = serow + jnp.sum(corr)
        av = jnp.clip(zf + a, -60.0, 60.0)
        lhs = jnp.exp(av) * serow
        rhs = uacc * jnp.maximum(se_y, 1e-30)
        acc = lhs > rhs

        nb0 = jnp.where(jnp.logical_and(acc, f0 > 0.0), 1.0 - x0, x0)
        nb1 = jnp.where(jnp.logical_and(acc, f1 > 0.0), 1.0 - x1, x1)

        plsc.store_scatter(xrow_v, [tix0], nb0, mask=lanes >= 0)
        plsc.store_scatter(xrow_v, [tix1], nb1, mask=lanes < 15)
        pltpu.sync_copy(xrow_v, out_hbm.at[r])
        return carry
    lax.fori_loop(0, _RPW, row_body, 0)


def _sc_call(x, pblk, bm, m, se, radius, u_accept, v, theta, U, gflat):
    f32 = jnp.float32
    i32 = jnp.int32
    mesh = plsc.VectorSubcoreMesh(core_axis_name="c", subcore_axis_name="s")
    kern = functools.partial(
        pl.kernel,
        mesh=mesh,
        out_type=jax.ShapeDtypeStruct((_B, _D), f32),
        scratch_types=[
            pltpu.VMEM((_D,), f32),
            pltpu.VMEM((_NBLK,), f32),
            pltpu.VMEM((2 * 16,), i32),
            pltpu.VMEM((2 * 16,), i32),
            pltpu.VMEM((_MAXR + 1, _W), f32),
            pltpu.VMEM((_POOL,), f32),
            pltpu.VMEM((_POOL,), i32),
            pltpu.VMEM((2 * 16,), i32),
            pltpu.VMEM((2 * 16,), f32),
            pltpu.VMEM((2 * 16,), i32),
            pltpu.VMEM((2 * 16,), i32),
            pltpu.VMEM((2 * 16,), f32),
            pltpu.VMEM((2 * 16,), f32),
            pltpu.VMEM((_MAXR + 1, 2 * _K), f32),
            pltpu.VMEM((_K,), f32),
            pltpu.VMEM((2 * 16,), f32),
            pltpu.VMEM((_B,), i32),
            pltpu.VMEM((_B,), f32),
            pltpu.VMEM((_B,), f32),
            pltpu.VMEM((_B,), f32),
            pltpu.SemaphoreType.DMA,
            pltpu.SemaphoreType.DMA,
        ],
        compiler_params=pltpu.CompilerParams(needs_layout_passes=False),
    )(_sc_body)
    return kern(x, pblk, bm, m, se, radius, u_accept, v, theta, U, gflat)


def kernel(x, theta, U, radius, gumbel, u_accept):
    f32 = jnp.float32
    x = x.astype(f32)
    th2 = theta.reshape(1, _D)

    v = pl.pallas_call(
        _acc_kernel,
        grid=(_ND,),
        in_specs=[
            pl.BlockSpec((_B, _DT), lambda j: (0, j)),
            pl.BlockSpec((_DT, _K), lambda j: (j, 0)),
        ],
        out_specs=pl.BlockSpec((_B, _K), lambda j: (0, 0)),
        out_shape=jax.ShapeDtypeStruct((_B, _K), f32),
        interpret=_INTERPRET,
    )(x, U)

    outs = pl.pallas_call(
        _score_kernel,
        grid=(_ND, _NR),
        in_specs=[
            pl.BlockSpec((_RB, _DT), lambda j, i: (i, j)),
            pl.BlockSpec((_RB, _DT), lambda j, i: (i, j)),
            pl.BlockSpec((_DT, _K), lambda j, i: (j, 0)),
            pl.BlockSpec((1, _DT), lambda j, i: (0, j)),
            pl.BlockSpec((_RB, _K), lambda j, i: (i, 0)),
        ],
        out_specs=[
            pl.BlockSpec((_RB, _DT), lambda j, i: (i, j)),
            pl.BlockSpec((1, _RB, _NBW), lambda j, i: (j, i, 0)),
            pl.BlockSpec((_RB, 1), lambda j, i: (i, 0)),
            pl.BlockSpec((_RB, 1), lambda j, i: (i, 0)),
        ],
        out_shape=[
            jax.ShapeDtypeStruct((_B, _D), f32),
            jax.ShapeDtypeStruct((_ND, _B, _NBW), f32),
            jax.ShapeDtypeStruct((_B, 1), f32),
            jax.ShapeDtypeStruct((_B, 1), f32),
        ],
        scratch_shapes=[
            pltpu.VMEM((_B, 1), f32),
            pltpu.VMEM((_B, 1), f32),
        ],
        interpret=_INTERPRET,
    )(x, gumbel, U, th2, v)
    pert, bm3, m, se = outs

    bm = bm3.transpose(1, 0, 2).reshape(_B, _NBLK)
    return _sc_call(
        x, pert.reshape(_B * _NBLK, _W), bm,
        m.reshape(_B), se.reshape(_B), radius.reshape(_B).astype(jnp.int32),
        u_accept, v, theta, U.reshape(_D // 2, 2 * _K),
        gumbel.reshape(_B * _D))


# speedup vs baseline: 6.4245x; 4.2645x over previous
import functools

import jax
import jax.numpy as jnp
from jax import lax
from jax.experimental import pallas as pl
from jax.experimental.pallas import tpu as pltpu
from jax.experimental.pallas import tpu_sc as plsc

_B, _D, _K = 128, 32768, 64
_DT = 2048
_ND = _D // _DT
_RB = 64
_NR = _B // _RB
_W = 128
_NBW = _DT // _W
_NBLK = _D // _W
_MAXR = 31
_NW = 32
_RPW = _B // _NW
_POOL = _MAXR * _W + 16
_NEG = -3.0e38

_INTERPRET = False


def _acc_kernel(x_ref, u_ref, v_ref):
    @pl.when(pl.program_id(0) == 0)
    def _():
        v_ref[...] = jnp.zeros_like(v_ref)
    v_ref[...] += jnp.dot(x_ref[...], u_ref[...],
                          preferred_element_type=jnp.float32)


def _score_kernel(x_ref, g_ref, u_ref, th_ref, v_ref,
                  pert_ref, bm_ref, m_ref, se_ref,
                  macc, seacc):
    j = pl.program_id(0)
    x = x_ref[...]
    grad = th_ref[...] + lax.dot_general(
        v_ref[...], u_ref[...], (((1,), (1,)), ((), ())),
        preferred_element_type=jnp.float32)
    s = (0.5 - x) * grad
    pert = s + g_ref[...]
    pert_ref[...] = pert
    bm_ref[...] = jnp.max(pert.reshape(_RB, _NBW, _W), axis=-1)[None]
    tm = jnp.max(s, axis=-1, keepdims=True)
    tse = jnp.sum(jnp.exp(s - tm), axis=-1, keepdims=True)
    row0 = pl.program_id(1) * _RB

    @pl.when(j == 0)
    def _():
        macc[pl.ds(row0, _RB), :] = tm
        seacc[pl.ds(row0, _RB), :] = tse

    @pl.when(j > 0)
    def _():
        m_old = macc[pl.ds(row0, _RB), :]
        se_old = seacc[pl.ds(row0, _RB), :]
        m_new = jnp.maximum(m_old, tm)
        macc[pl.ds(row0, _RB), :] = m_new
        seacc[pl.ds(row0, _RB), :] = (se_old * jnp.exp(m_old - m_new)
                                      + tse * jnp.exp(tm - m_new))

    @pl.when(j == _ND - 1)
    def _():
        m_ref[...] = macc[pl.ds(row0, _RB), :]
        se_ref[...] = seacc[pl.ds(row0, _RB), :]


def _sc_body(x_hbm, pblk_hbm, bm_hbm, m_hbm, se_hbm, rad_hbm, uac_hbm,
             v_hbm, th_hbm, utab_hbm, gf_hbm,
             out_hbm,
             xrow_v, bm_v, blkid_v, cbase_v, blocks_v, pool_v, poolix_v,
             topix_v, topval_v, gidx_v, uidx_v, gsel_v, thsel_v, usel_v,
             vrow_v, fd_v, rad_v, uac_v, m_v, se_v, sem, sem2):
    i32 = jnp.int32
    f32 = jnp.float32
    lanes = lax.iota(i32, 16)
    zi = jnp.zeros((16,), i32)
    zf = jnp.zeros((16,), f32)

    wid = lax.axis_index("s") * 2 + lax.axis_index("c")
    base = wid * _RPW

    pltpu.sync_copy(rad_hbm, rad_v)
    pltpu.sync_copy(uac_hbm, uac_v)
    pltpu.sync_copy(m_hbm, m_v)
    pltpu.sync_copy(se_hbm, se_v)

    def row_body(rr, carry):
        r = base + rr
        cpx = pltpu.async_copy(x_hbm.at[r], xrow_v, sem2)
        pltpu.sync_copy(bm_hbm.at[r], bm_v)
        pltpu.sync_copy(v_hbm.at[r], vrow_v)

        blkid_v[pl.ds(0, 16)] = zi
        blkid_v[pl.ds(16, 16)] = zi
        cbase_v[pl.ds(0, 16)] = zi
        cbase_v[pl.ds(16, 16)] = zi

        def ext_blk(j, _t):
            rv = jnp.full((16,), _NEG, f32)
            for k in range(16):
                rv = jnp.maximum(rv, bm_v[pl.ds(16 * k, 16)])
            gmax = jnp.max(rv)
            posv = jnp.full((16,), -1, i32)
            for k in range(16):
                pv = bm_v[pl.ds(16 * k, 16)]
                posv = jnp.maximum(
                    posv, jnp.where(pv == gmax, lanes + 16 * k, -1))
            pos = jnp.max(posv)
            plsc.store_scatter(bm_v, [zi + pos], zf + _NEG,
                               mask=lanes == 0)
            plsc.store_scatter(blkid_v, [zi + j], zi + (r * _NBLK + pos),
                               mask=lanes == 0)
            plsc.store_scatter(cbase_v, [zi + j], zi + pos * _W,
                               mask=lanes == 0)
            return gmax
        t31 = lax.fori_loop(0, _MAXR, ext_blk, _NEG)

        pltpu.async_copy(pblk_hbm.at[blkid_v], blocks_v, sem).wait()

        t31v = zf + t31

        def comp_blk(jj, cur):
            cb = plsc.load_gather(cbase_v, [zi + jj])
            for k in range(8):
                pv = blocks_v[jj, pl.ds(16 * k, 16)]
                msk = pv >= t31v
                plsc.store_compressed(pool_v.at[pl.ds(cur, 16)], pv, mask=msk)
                plsc.store_compressed(poolix_v.at[pl.ds(cur, 16)],
                                      cb + (16 * k) + lanes, mask=msk)
                cur = cur + jnp.max(
                    plsc.all_reduce_population_count(msk))
            return cur
        psize = lax.fori_loop(0, _MAXR, comp_blk, jnp.int32(0))
        pool_v[pl.ds(psize, 16)] = zf + _NEG
        poolix_v[pl.ds(psize, 16)] = zi

        nv = (psize + 15) // 16
        topix_v[pl.ds(0, 16)] = zi
        topix_v[pl.ds(16, 16)] = zi

        def ext_pool(j, _u):
            def scan1(k, rv):
                return jnp.maximum(rv, pool_v[pl.ds(16 * k, 16)])
            rv = lax.fori_loop(0, nv, scan1, jnp.full((16,), _NEG, f32))
            gmax = jnp.max(rv)

            def scan2(k, c):
                posv, idxv = c
                pv = pool_v[pl.ds(16 * k, 16)]
                hit = pv == gmax
                posv = jnp.maximum(
                    posv, jnp.where(hit, lanes + 16 * k, -1))
                idxv = jnp.maximum(
                    idxv, jnp.where(hit, poolix_v[pl.ds(16 * k, 16)], -1))
                return posv, idxv
            posv, idxv = lax.fori_loop(
                0, nv, scan2,
                (jnp.full((16,), -1, i32), jnp.full((16,), -1, i32)))
            pos = jnp.max(posv)
            col = jnp.max(idxv)
            plsc.store_scatter(pool_v, [zi + pos], zf + _NEG,
                               mask=lanes == 0)
            plsc.store_scatter(topix_v, [zi + j], zi + col,
                               mask=lanes == 0)
            plsc.store_scatter(topval_v, [zi + j], zf + gmax,
                               mask=lanes == 0)
            return 0
        lax.fori_loop(0, _MAXR, ext_pool, 0)

        tix0 = topix_v[pl.ds(0, 16)]
        tix1 = topix_v[pl.ds(16, 16)]
        gidx_v[pl.ds(0, 16)] = tix0 + r * _D
        gidx_v[pl.ds(16, 16)] = tix1 + r * _D
        uidx_v[pl.ds(0, 16)] = tix0 // 2
        uidx_v[pl.ds(16, 16)] = tix1 // 2
        pltpu.async_copy(gf_hbm.at[gidx_v], gsel_v, sem).wait()
        pltpu.async_copy(th_hbm.at[topix_v], thsel_v, sem).wait()
        pltpu.async_copy(utab_hbm.at[uidx_v], usel_v, sem).wait()
        cpx.wait()
        x0 = plsc.load_gather(xrow_v, [tix0])
        x1 = plsc.load_gather(xrow_v, [tix1])

        rad = plsc.load_gather(rad_v, [zi + r])
        uacc = plsc.load_gather(uac_v, [zi + r])
        mrow = plsc.load_gather(m_v, [zi + r])
        serow = plsc.load_gather(se_v, [zi + r])

        tv0 = topval_v[pl.ds(0, 16)]
        tv1 = topval_v[pl.ds(16, 16)]
        s0 = tv0 - gsel_v[pl.ds(0, 16)]
        s1 = tv1 - gsel_v[pl.ds(16, 16)]
        d0 = 1.0 - 2.0 * x0
        d1 = 1.0 - 2.0 * x1
        f0 = jnp.where(lanes < rad, 1.0, 0.0)
        f1 = jnp.where(lanes + 16 < rad, 1.0, 0.0)
        fd0 = f0 * d0
        fd1 = f1 * d1
        fd_v[pl.ds(0, 16)] = fd0
        fd_v[pl.ds(16, 16)] = fd1

        def wacc_body(j, wc):
            fdj = plsc.load_gather(fd_v, [zi + j])
            tj = jnp.max(plsc.load_gather(topix_v, [zi + j]))
            half = (tj % 2) * 64
            return tuple(
                wc[c] + fdj * usel_v[j, pl.ds(half + 16 * c, 16)]
                for c in range(4))
        w = lax.fori_loop(0, _MAXR, wacc_body, (zf, zf, zf, zf))

        dth = jnp.sum(fd0 * thsel_v[pl.ds(0, 16)]
                      + fd1 * thsel_v[pl.ds(16, 16)])
        vw = jnp.float32(0.0)
        ww = jnp.float32(0.0)
        for c in range(4):
            vv = vrow_v[pl.ds(16 * c, 16)]
            vw = vw + jnp.sum(vv * w[c])
            ww = ww + jnp.sum(w[c] * w[c])
        dscore = dth + vw + 0.5 * ww
        sum_s = jnp.sum(f0 * s0 + f1 * s1)
        a = dscore - 2.0 * sum_s

        corr = (f0 * (jnp.exp(-s0 - mrow) - jnp.exp(s0 - mrow))
                + f1 * (jnp.exp(-s1 - mrow) - jnp.exp(s1 - mrow)))
        se_y = serow + jnp.sum(corr)
        av = jnp.clip(zf + a, -60.0, 60.0)
        lhs = jnp.exp(av) * serow
        rhs = uacc * jnp.maximum(se_y, 1e-30)
        acc = lhs > rhs

        nb0 = jnp.where(jnp.logical_and(acc, f0 > 0.0), 1.0 - x0, x0)
        nb1 = jnp.where(jnp.logical_and(acc, f1 > 0.0), 1.0 - x1, x1)

        plsc.store_scatter(xrow_v, [tix0], nb0, mask=lanes >= 0)
        plsc.store_scatter(xrow_v, [tix1], nb1, mask=lanes < 15)
        pltpu.sync_copy(xrow_v, out_hbm.at[r])
        return carry
    lax.fori_loop(0, _RPW, row_body, 0)


def _sc_call(x, pblk, bm, m, se, radius, u_accept, v, theta, U, gflat):
    f32 = jnp.float32
    i32 = jnp.int32
    mesh = plsc.VectorSubcoreMesh(core_axis_name="c", subcore_axis_name="s")
    kern = functools.partial(
        pl.kernel,
        mesh=mesh,
        out_type=jax.ShapeDtypeStruct((_B, _D), f32),
        scratch_types=[
            pltpu.VMEM((_D,), f32),
            pltpu.VMEM((_NBLK,), f32),
            pltpu.VMEM((2 * 16,), i32),
            pltpu.VMEM((2 * 16,), i32),
            pltpu.VMEM((_MAXR + 1, _W), f32),
            pltpu.VMEM((_POOL,), f32),
            pltpu.VMEM((_POOL,), i32),
            pltpu.VMEM((2 * 16,), i32),
            pltpu.VMEM((2 * 16,), f32),
            pltpu.VMEM((2 * 16,), i32),
            pltpu.VMEM((2 * 16,), i32),
            pltpu.VMEM((2 * 16,), f32),
            pltpu.VMEM((2 * 16,), f32),
            pltpu.VMEM((_MAXR + 1, 2 * _K), f32),
            pltpu.VMEM((_K,), f32),
            pltpu.VMEM((2 * 16,), f32),
            pltpu.VMEM((_B,), i32),
            pltpu.VMEM((_B,), f32),
            pltpu.VMEM((_B,), f32),
            pltpu.VMEM((_B,), f32),
            pltpu.SemaphoreType.DMA,
            pltpu.SemaphoreType.DMA,
        ],
        compiler_params=pltpu.CompilerParams(needs_layout_passes=False),
    )(_sc_body)
    return kern(x, pblk, bm, m, se, radius, u_accept, v, theta, U, gflat)


def kernel(x, theta, U, radius, gumbel, u_accept):
    f32 = jnp.float32
    x = x.astype(f32)
    th2 = theta.reshape(1, _D)

    v = pl.pallas_call(
        _acc_kernel,
        grid=(_ND,),
        in_specs=[
            pl.BlockSpec((_B, _DT), lambda j: (0, j)),
            pl.BlockSpec((_DT, _K), lambda j: (j, 0)),
        ],
        out_specs=pl.BlockSpec((_B, _K), lambda j: (0, 0)),
        out_shape=jax.ShapeDtypeStruct((_B, _K), f32),
        interpret=_INTERPRET,
    )(x, U)

    outs = pl.pallas_call(
        _score_kernel,
        grid=(_ND, _NR),
        in_specs=[
            pl.BlockSpec((_RB, _DT), lambda j, i: (i, j)),
            pl.BlockSpec((_RB, _DT), lambda j, i: (i, j)),
            pl.BlockSpec((_DT, _K), lambda j, i: (j, 0)),
            pl.BlockSpec((1, _DT), lambda j, i: (0, j)),
            pl.BlockSpec((_RB, _K), lambda j, i: (i, 0)),
        ],
        out_specs=[
            pl.BlockSpec((_RB, _DT), lambda j, i: (i, j)),
            pl.BlockSpec((1, _RB, _NBW), lambda j, i: (j, i, 0)),
            pl.BlockSpec((_RB, 1), lambda j, i: (i, 0)),
            pl.BlockSpec((_RB, 1), lambda j, i: (i, 0)),
        ],
        out_shape=[
            jax.ShapeDtypeStruct((_B, _D), f32),
            jax.ShapeDtypeStruct((_ND, _B, _NBW), f32),
            jax.ShapeDtypeStruct((_B, 1), f32),
            jax.ShapeDtypeStruct((_B, 1), f32),
        ],
        scratch_shapes=[
            pltpu.VMEM((_B, 1), f32),
            pltpu.VMEM((_B, 1), f32),
        ],
        interpret=_INTERPRET,
    )(x, gumbel, U, th2, v)
    pert, bm3, m, se = outs

    bm = bm3.transpose(1, 0, 2).reshape(_B, _NBLK)
    return _sc_call(
        x, pert.reshape(_B * _NBLK, _W), bm,
        m.reshape(_B), se.reshape(_B), radius.reshape(_B).astype(jnp.int32),
        u_accept, v, theta, U.reshape(_D // 2, 2 * _K),
        gumbel.reshape(_B * _D))


# speedup vs baseline: 7.0012x; 1.0898x over previous
import functools

import jax
import jax.numpy as jnp
from jax import lax
from jax.experimental import pallas as pl
from jax.experimental.pallas import tpu as pltpu
from jax.experimental.pallas import tpu_sc as plsc

_B, _D, _K = 128, 32768, 64
_DT = 4096
_ND = _D // _DT
_RB = 64
_NR = _B // _RB
_W = 128
_NBW = _DT // _W
_NBLK = _D // _W
_MAXR = 31
_NW = 32
_RPW = _B // _NW
_POOL = _MAXR * _W + 16
_NEG = -3.0e38

_INTERPRET = False


def _acc_kernel(x_ref, u_ref, v_ref):
    @pl.when(pl.program_id(0) == 0)
    def _():
        v_ref[...] = jnp.zeros_like(v_ref)
    v_ref[...] += jnp.dot(x_ref[...], u_ref[...],
                          preferred_element_type=jnp.float32)


def _score_kernel(x_ref, g_ref, u_ref, th_ref, v_ref,
                  pert_ref, bm_ref, m_ref, se_ref,
                  macc, seacc):
    j = pl.program_id(0)
    x = x_ref[...]
    grad = th_ref[...] + lax.dot_general(
        v_ref[...], u_ref[...], (((1,), (1,)), ((), ())),
        preferred_element_type=jnp.float32)
    s = (0.5 - x) * grad
    pert = s + g_ref[...]
    pert_ref[...] = pert
    tm = jnp.max(s, axis=-1, keepdims=True)
    tse = jnp.sum(jnp.exp(s - tm), axis=-1, keepdims=True)
    row0 = pl.program_id(1) * _RB
    bm_ref[...] = jnp.max(pert.reshape(_RB, _NBW, _W), axis=-1)[None]

    @pl.when(j == 0)
    def _():
        macc[pl.ds(row0, _RB), :] = tm
        seacc[pl.ds(row0, _RB), :] = tse

    @pl.when(j > 0)
    def _():
        m_old = macc[pl.ds(row0, _RB), :]
        se_old = seacc[pl.ds(row0, _RB), :]
        m_new = jnp.maximum(m_old, tm)
        macc[pl.ds(row0, _RB), :] = m_new
        seacc[pl.ds(row0, _RB), :] = (se_old * jnp.exp(m_old - m_new)
                                      + tse * jnp.exp(tm - m_new))

    @pl.when(j == _ND - 1)
    def _():
        m_ref[...] = macc[pl.ds(row0, _RB), :]
        se_ref[...] = seacc[pl.ds(row0, _RB), :]


def _sc_body(x_hbm, pblk_hbm, bm_hbm, m_hbm, se_hbm, rad_hbm, uac_hbm,
             v_hbm, th_hbm, utab_hbm, gf_hbm,
             out_hbm,
             xrow_v, bm_v, blkid_v, cbase_v, blocks_v, pool_v, poolix_v,
             topix_v, topval_v, gidx_v, uidx_v, gsel_v, thsel_v, usel_v,
             vrow_v, fd_v, rad_v, uac_v, m_v, se_v, sem, sem2):
    i32 = jnp.int32
    f32 = jnp.float32
    lanes = lax.iota(i32, 16)
    zi = jnp.zeros((16,), i32)
    zf = jnp.zeros((16,), f32)

    wid = lax.axis_index("s") * 2 + lax.axis_index("c")
    base = wid * _RPW

    pltpu.sync_copy(rad_hbm, rad_v)
    pltpu.sync_copy(uac_hbm, uac_v)
    pltpu.sync_copy(m_hbm, m_v)
    pltpu.sync_copy(se_hbm, se_v)

    def row_body(rr, carry):
        r = base + rr
        cpx = pltpu.async_copy(x_hbm.at[r], xrow_v, sem2)
        pltpu.sync_copy(bm_hbm.at[:, r], bm_v)
        pltpu.sync_copy(v_hbm.at[r], vrow_v)

        blkid_v[pl.ds(0, 16)] = zi
        blkid_v[pl.ds(16, 16)] = zi
        cbase_v[pl.ds(0, 16)] = zi
        cbase_v[pl.ds(16, 16)] = zi

        def ext_blk(j, _t):
            rv = jnp.full((16,), _NEG, f32)
            for k in range(16):
                rv = jnp.maximum(
                    rv, bm_v[k // 2, pl.ds((k % 2) * 16, 16)])
            gmax = jnp.max(rv)
            posv = jnp.full((16,), -1, i32)
            for k in range(16):
                pv = bm_v[k // 2, pl.ds((k % 2) * 16, 16)]
                posv = jnp.maximum(
                    posv, jnp.where(pv == gmax, lanes + 16 * k, -1))
            pos = jnp.max(posv)
            plsc.store_scatter(bm_v, [zi + pos // _NBW, zi + pos % _NBW],
                               zf + _NEG, mask=lanes == 0)
            plsc.store_scatter(blkid_v, [zi + j], zi + (r * _NBLK + pos),
                               mask=lanes == 0)
            plsc.store_scatter(cbase_v, [zi + j], zi + pos * _W,
                               mask=lanes == 0)
            return gmax
        t31 = lax.fori_loop(0, _MAXR, ext_blk, _NEG)

        pltpu.async_copy(pblk_hbm.at[blkid_v], blocks_v, sem).wait()

        t31v = zf + t31

        def comp_blk(jj, cur):
            cb = plsc.load_gather(cbase_v, [zi + jj])
            for k in range(8):
                pv = blocks_v[jj, pl.ds(16 * k, 16)]
                msk = pv >= t31v
                plsc.store_compressed(pool_v.at[pl.ds(cur, 16)], pv, mask=msk)
                plsc.store_compressed(poolix_v.at[pl.ds(cur, 16)],
                                      cb + (16 * k) + lanes, mask=msk)
                cur = cur + jnp.max(
                    plsc.all_reduce_population_count(msk))
            return cur
        psize = lax.fori_loop(0, _MAXR, comp_blk, jnp.int32(0))
        pool_v[pl.ds(psize, 16)] = zf + _NEG
        poolix_v[pl.ds(psize, 16)] = zi

        nv = (psize + 15) // 16
        topix_v[pl.ds(0, 16)] = zi
        topix_v[pl.ds(16, 16)] = zi

        def ext_pool(j, _u):
            def scan1(k, rv):
                return jnp.maximum(rv, pool_v[pl.ds(16 * k, 16)])
            rv = lax.fori_loop(0, nv, scan1, jnp.full((16,), _NEG, f32))
            gmax = jnp.max(rv)

            def scan2(k, c):
                posv, idxv = c
                pv = pool_v[pl.ds(16 * k, 16)]
                hit = pv == gmax
                posv = jnp.maximum(
                    posv, jnp.where(hit, lanes + 16 * k, -1))
                idxv = jnp.maximum(
                    idxv, jnp.where(hit, poolix_v[pl.ds(16 * k, 16)], -1))
                return posv, idxv
            posv, idxv = lax.fori_loop(
                0, nv, scan2,
                (jnp.full((16,), -1, i32), jnp.full((16,), -1, i32)))
            pos = jnp.max(posv)
            col = jnp.max(idxv)
            plsc.store_scatter(pool_v, [zi + pos], zf + _NEG,
                               mask=lanes == 0)
            plsc.store_scatter(topix_v, [zi + j], zi + col,
                               mask=lanes == 0)
            plsc.store_scatter(topval_v, [zi + j], zf + gmax,
                               mask=lanes == 0)
            return 0
        lax.fori_loop(0, _MAXR, ext_pool, 0)

        tix0 = topix_v[pl.ds(0, 16)]
        tix1 = topix_v[pl.ds(16, 16)]
        gidx_v[pl.ds(0, 16)] = tix0 + r * _D
        gidx_v[pl.ds(16, 16)] = tix1 + r * _D
        uidx_v[pl.ds(0, 16)] = tix0 // 2
        uidx_v[pl.ds(16, 16)] = tix1 // 2
        pltpu.async_copy(gf_hbm.at[gidx_v], gsel_v, sem).wait()
        pltpu.async_copy(th_hbm.at[topix_v], thsel_v, sem).wait()
        pltpu.async_copy(utab_hbm.at[uidx_v], usel_v, sem).wait()
        cpx.wait()
        x0 = plsc.load_gather(xrow_v, [tix0])
        x1 = plsc.load_gather(xrow_v, [tix1])

        rad = plsc.load_gather(rad_v, [zi + r])
        uacc = plsc.load_gather(uac_v, [zi + r])
        mrow = plsc.load_gather(m_v, [zi + r])
        serow = plsc.load_gather(se_v, [zi + r])

        tv0 = topval_v[pl.ds(0, 16)]
        tv1 = topval_v[pl.ds(16, 16)]
        s0 = tv0 - gsel_v[pl.ds(0, 16)]
        s1 = tv1 - gsel_v[pl.ds(16, 16)]
        d0 = 1.0 - 2.0 * x0
        d1 = 1.0 - 2.0 * x1
        f0 = jnp.where(lanes < rad, 1.0, 0.0)
        f1 = jnp.where(lanes + 16 < rad, 1.0, 0.0)
        fd0 = f0 * d0
        fd1 = f1 * d1
        fd_v[pl.ds(0, 16)] = fd0
        fd_v[pl.ds(16, 16)] = fd1

        def wacc_body(j, wc):
            fdj = plsc.load_gather(fd_v, [zi + j])
            tj = jnp.max(plsc.load_gather(topix_v, [zi + j]))
            half = (tj % 2) * 64
            return tuple(
                wc[c] + fdj * usel_v[j, pl.ds(half + 16 * c, 16)]
                for c in range(4))
        w = lax.fori_loop(0, _MAXR, wacc_body, (zf, zf, zf, zf))

        dth = jnp.sum(fd0 * thsel_v[pl.ds(0, 16)]
                      + fd1 * thsel_v[pl.ds(16, 16)])
        vw = jnp.float32(0.0)
        ww = jnp.float32(0.0)
        for c in range(4):
            vv = vrow_v[pl.ds(16 * c, 16)]
            vw = vw + jnp.sum(vv * w[c])
            ww = ww + jnp.sum(w[c] * w[c])
        dscore = dth + vw + 0.5 * ww
        sum_s = jnp.sum(f0 * s0 + f1 * s1)
        a = dscore - 2.0 * sum_s

        corr = (f0 * (jnp.exp(-s0 - mrow) - jnp.exp(s0 - mrow))
                + f1 * (jnp.exp(-s1 - mrow) - jnp.exp(s1 - mrow)))
        se_y = serow + jnp.sum(corr)
        av = jnp.clip(zf + a, -60.0, 60.0)
        lhs = jnp.exp(av) * serow
        rhs = uacc * jnp.maximum(se_y, 1e-30)
        acc = lhs > rhs

        nb0 = jnp.where(jnp.logical_and(acc, f0 > 0.0), 1.0 - x0, x0)
        nb1 = jnp.where(jnp.logical_and(acc, f1 > 0.0), 1.0 - x1, x1)

        plsc.store_scatter(xrow_v, [tix0], nb0, mask=lanes >= 0)
        plsc.store_scatter(xrow_v, [tix1], nb1, mask=lanes < 15)
        pltpu.sync_copy(xrow_v, out_hbm.at[r])
        return carry
    lax.fori_loop(0, _RPW, row_body, 0)


def _sc_call(x, pblk, bm, m, se, radius, u_accept, v, theta, U, gflat):
    f32 = jnp.float32
    i32 = jnp.int32
    mesh = plsc.VectorSubcoreMesh(core_axis_name="c", subcore_axis_name="s")
    kern = functools.partial(
        pl.kernel,
        mesh=mesh,
        out_type=jax.ShapeDtypeStruct((_B, _D), f32),
        scratch_types=[
            pltpu.VMEM((_D,), f32),
            pltpu.VMEM((_ND, _NBW), f32),
            pltpu.VMEM((2 * 16,), i32),
            pltpu.VMEM((2 * 16,), i32),
            pltpu.VMEM((_MAXR + 1, _W), f32),
            pltpu.VMEM((_POOL,), f32),
            pltpu.VMEM((_POOL,), i32),
            pltpu.VMEM((2 * 16,), i32),
            pltpu.VMEM((2 * 16,), f32),
            pltpu.VMEM((2 * 16,), i32),
            pltpu.VMEM((2 * 16,), i32),
            pltpu.VMEM((2 * 16,), f32),
            pltpu.VMEM((2 * 16,), f32),
            pltpu.VMEM((_MAXR + 1, 2 * _K), f32),
            pltpu.VMEM((_K,), f32),
            pltpu.VMEM((2 * 16,), f32),
            pltpu.VMEM((_B,), i32),
            pltpu.VMEM((_B,), f32),
            pltpu.VMEM((_B,), f32),
            pltpu.VMEM((_B,), f32),
            pltpu.SemaphoreType.DMA,
            pltpu.SemaphoreType.DMA,
        ],
        compiler_params=pltpu.CompilerParams(needs_layout_passes=False),
    )(_sc_body)
    return kern(x, pblk, bm, m, se, radius, u_accept, v, theta, U, gflat)


def kernel(x, theta, U, radius, gumbel, u_accept):
    f32 = jnp.float32
    x = x.astype(f32)
    th2 = theta.reshape(1, _D)

    v = pl.pallas_call(
        _acc_kernel,
        grid=(_ND,),
        in_specs=[
            pl.BlockSpec((_B, _DT), lambda j: (0, j)),
            pl.BlockSpec((_DT, _K), lambda j: (j, 0)),
        ],
        out_specs=pl.BlockSpec((_B, _K), lambda j: (0, 0)),
        out_shape=jax.ShapeDtypeStruct((_B, _K), f32),
        interpret=_INTERPRET,
    )(x, U)

    outs = pl.pallas_call(
        _score_kernel,
        grid=(_ND, _NR),
        in_specs=[
            pl.BlockSpec((_RB, _DT), lambda j, i: (i, j)),
            pl.BlockSpec((_RB, _DT), lambda j, i: (i, j)),
            pl.BlockSpec((_DT, _K), lambda j, i: (j, 0)),
            pl.BlockSpec((1, _DT), lambda j, i: (0, j)),
            pl.BlockSpec((_RB, _K), lambda j, i: (i, 0)),
        ],
        out_specs=[
            pl.BlockSpec((_RB, _DT), lambda j, i: (i, j)),
            pl.BlockSpec((1, _RB, _NBW), lambda j, i: (j, i, 0)),
            pl.BlockSpec((_RB, 1), lambda j, i: (i, 0)),
            pl.BlockSpec((_RB, 1), lambda j, i: (i, 0)),
        ],
        out_shape=[
            jax.ShapeDtypeStruct((_B, _D), f32),
            jax.ShapeDtypeStruct((_ND, _B, _NBW), f32),
            jax.ShapeDtypeStruct((_B, 1), f32),
            jax.ShapeDtypeStruct((_B, 1), f32),
        ],
        scratch_shapes=[
            pltpu.VMEM((_B, 1), f32),
            pltpu.VMEM((_B, 1), f32),
        ],
        interpret=_INTERPRET,
    )(x, gumbel, U, th2, v)
    pert, bm3, m, se = outs

    return _sc_call(
        x, pert.reshape(_B * _NBLK, _W), bm3,
        m.reshape(_B), se.reshape(_B), radius.reshape(_B).astype(jnp.int32),
        u_accept, v, theta, U.reshape(_D // 2, 2 * _K),
        gumbel.reshape(_B * _D))


# speedup vs baseline: 7.3196x; 1.0455x over previous
import functools

import jax
import jax.numpy as jnp
from jax import lax
from jax.experimental import pallas as pl
from jax.experimental.pallas import tpu as pltpu
from jax.experimental.pallas import tpu_sc as plsc

_B, _D, _K = 128, 32768, 64
_DT = 4096
_ND = _D // _DT
_RB = 64
_NR = _B // _RB
_W = 128
_NBW = _DT // _W
_NBLK = _D // _W
_MAXR = 31
_NW = 32
_RPW = _B // _NW
_POOL = _MAXR * _W + 16
_NEG = -3.0e38

_INTERPRET = False


def _acc_kernel(x_ref, u_ref, v_ref):
    @pl.when(pl.program_id(0) == 0)
    def _():
        v_ref[...] = jnp.zeros_like(v_ref)
    v_ref[...] += jnp.dot(x_ref[...], u_ref[...],
                          preferred_element_type=jnp.float32)


def _score_kernel(x_ref, g_ref, u_ref, th_ref, v_ref,
                  pert_ref, bm_ref, m_ref, se_ref,
                  macc, seacc):
    j = pl.program_id(0)
    x = x_ref[...]
    grad = th_ref[...] + lax.dot_general(
        v_ref[...], u_ref[...], (((1,), (1,)), ((), ())),
        preferred_element_type=jnp.float32)
    s = (0.5 - x) * grad
    pert = s + g_ref[...]
    pert_ref[...] = pert
    tm = jnp.max(s, axis=-1, keepdims=True)
    tse = jnp.sum(jnp.exp(s - tm), axis=-1, keepdims=True)
    row0 = pl.program_id(1) * _RB
    bm_ref[...] = jnp.max(pert.reshape(_RB, _NBW, _W), axis=-1)[None]

    @pl.when(j == 0)
    def _():
        macc[pl.ds(row0, _RB), :] = tm
        seacc[pl.ds(row0, _RB), :] = tse

    @pl.when(j > 0)
    def _():
        m_old = macc[pl.ds(row0, _RB), :]
        se_old = seacc[pl.ds(row0, _RB), :]
        m_new = jnp.maximum(m_old, tm)
        macc[pl.ds(row0, _RB), :] = m_new
        seacc[pl.ds(row0, _RB), :] = (se_old * jnp.exp(m_old - m_new)
                                      + tse * jnp.exp(tm - m_new))

    @pl.when(j == _ND - 1)
    def _():
        m_ref[...] = macc[pl.ds(row0, _RB), :]
        se_ref[...] = seacc[pl.ds(row0, _RB), :]


def _sc_body(x_hbm, pblk_hbm, bm_hbm, m_hbm, se_hbm, rad_hbm, uac_hbm,
             v_hbm, th_hbm, utab_hbm, gf_hbm,
             out_hbm,
             xrow_v, bm_v, blkid_v, cbase_v, blocks_v, pool_v, poolix_v,
             topix_v, topval_v, gidx_v, uidx_v, gsel_v, thsel_v, usel_v,
             vrow_v, fd_v, rad_v, uac_v, m_v, se_v, sem, sem2):
    i32 = jnp.int32
    f32 = jnp.float32
    lanes = lax.iota(i32, 16)
    zi = jnp.zeros((16,), i32)
    zf = jnp.zeros((16,), f32)

    def sc0(vec):
        return lax.squeeze(lax.slice(vec, (0,), (1,)), (0,))

    wid = lax.axis_index("s") * 2 + lax.axis_index("c")
    base = wid * _RPW

    pltpu.sync_copy(rad_hbm, rad_v)
    pltpu.sync_copy(uac_hbm, uac_v)
    pltpu.sync_copy(m_hbm, m_v)
    pltpu.sync_copy(se_hbm, se_v)

    pltpu.async_copy(x_hbm.at[base], xrow_v, sem2)

    def row_body(rr, carry):
        r = base + rr
        pltpu.sync_copy(bm_hbm.at[:, r], bm_v)
        pltpu.sync_copy(v_hbm.at[r], vrow_v)

        blkid_v[pl.ds(0, 16)] = zi
        blkid_v[pl.ds(16, 16)] = zi
        cbase_v[pl.ds(0, 16)] = zi
        cbase_v[pl.ds(16, 16)] = zi

        def ext_blk(j, _t):
            rv = jnp.full((16,), _NEG, f32)
            for k in range(16):
                rv = jnp.maximum(
                    rv, bm_v[k // 2, pl.ds((k % 2) * 16, 16)])
            gmax = jnp.max(rv)
            posv = jnp.full((16,), -1, i32)
            for k in range(16):
                pv = bm_v[k // 2, pl.ds((k % 2) * 16, 16)]
                posv = jnp.maximum(
                    posv, jnp.where(pv == gmax, lanes + 16 * k, -1))
            pos = jnp.max(posv)
            plsc.store_scatter(bm_v, [zi + pos // _NBW, zi + pos % _NBW],
                               zf + _NEG, mask=lanes == 0)
            plsc.store_scatter(blkid_v, [zi + j], zi + (r * _NBLK + pos),
                               mask=lanes == 0)
            plsc.store_scatter(cbase_v, [zi + j], zi + pos * _W,
                               mask=lanes == 0)
            return gmax
        t31 = lax.fori_loop(0, _MAXR, ext_blk, _NEG)

        pltpu.async_copy(pblk_hbm.at[blkid_v], blocks_v, sem).wait()

        t31v = zf + t31

        def comp_blk(jj, cur):
            cb = plsc.load_gather(cbase_v, [zi + jj])
            for k in range(8):
                pv = blocks_v[jj, pl.ds(16 * k, 16)]
                msk = pv >= t31v
                plsc.store_compressed(pool_v.at[pl.ds(cur, 16)], pv, mask=msk)
                plsc.store_compressed(poolix_v.at[pl.ds(cur, 16)],
                                      cb + (16 * k) + lanes, mask=msk)
                cur = cur + sc0(plsc.all_reduce_population_count(msk))
            return cur
        psize = lax.fori_loop(0, _MAXR, comp_blk, jnp.int32(0))
        pool_v[pl.ds(psize, 16)] = zf + _NEG
        poolix_v[pl.ds(psize, 16)] = zi

        nv = (psize + 15) // 16
        topix_v[pl.ds(0, 16)] = zi
        topix_v[pl.ds(16, 16)] = zi

        def ext_pool(j, _u):
            def scan1(k, rv):
                return jnp.maximum(rv, pool_v[pl.ds(16 * k, 16)])
            rv = lax.fori_loop(0, nv, scan1, jnp.full((16,), _NEG, f32))
            gmax = jnp.max(rv)

            def scan2(k, c):
                posv, idxv = c
                pv = pool_v[pl.ds(16 * k, 16)]
                hit = pv == gmax
                posv = jnp.maximum(
                    posv, jnp.where(hit, lanes + 16 * k, -1))
                idxv = jnp.maximum(
                    idxv, jnp.where(hit, poolix_v[pl.ds(16 * k, 16)], -1))
                return posv, idxv
            posv, idxv = lax.fori_loop(
                0, nv, scan2,
                (jnp.full((16,), -1, i32), jnp.full((16,), -1, i32)))
            pos = jnp.max(posv)
            col = jnp.max(idxv)
            plsc.store_scatter(pool_v, [zi + pos], zf + _NEG,
                               mask=lanes == 0)
            plsc.store_scatter(topix_v, [zi + j], zi + col,
                               mask=lanes == 0)
            plsc.store_scatter(topval_v, [zi + j], zf + gmax,
                               mask=lanes == 0)
            return 0
        lax.fori_loop(0, _MAXR, ext_pool, 0)

        tix0 = topix_v[pl.ds(0, 16)]
        tix1 = topix_v[pl.ds(16, 16)]
        gidx_v[pl.ds(0, 16)] = tix0 + r * _D
        gidx_v[pl.ds(16, 16)] = tix1 + r * _D
        uidx_v[pl.ds(0, 16)] = tix0 // 2
        uidx_v[pl.ds(16, 16)] = tix1 // 2
        cg = pltpu.async_copy(gf_hbm.at[gidx_v], gsel_v, sem)
        ct = pltpu.async_copy(th_hbm.at[topix_v], thsel_v, sem)
        cu = pltpu.async_copy(utab_hbm.at[uidx_v], usel_v, sem)
        pltpu.make_async_copy(x_hbm.at[r], xrow_v, sem2).wait()
        x0 = plsc.load_gather(xrow_v, [tix0])
        x1 = plsc.load_gather(xrow_v, [tix1])
        cg.wait()
        ct.wait()
        cu.wait()

        rad = plsc.load_gather(rad_v, [zi + r])
        uacc = plsc.load_gather(uac_v, [zi + r])
        mrow = plsc.load_gather(m_v, [zi + r])
        serow = plsc.load_gather(se_v, [zi + r])

        tv0 = topval_v[pl.ds(0, 16)]
        tv1 = topval_v[pl.ds(16, 16)]
        s0 = tv0 - gsel_v[pl.ds(0, 16)]
        s1 = tv1 - gsel_v[pl.ds(16, 16)]
        d0 = 1.0 - 2.0 * x0
        d1 = 1.0 - 2.0 * x1
        f0 = jnp.where(lanes < rad, 1.0, 0.0)
        f1 = jnp.where(lanes + 16 < rad, 1.0, 0.0)
        fd0 = f0 * d0
        fd1 = f1 * d1
        fd_v[pl.ds(0, 16)] = fd0
        fd_v[pl.ds(16, 16)] = fd1

        def wacc_body(j, wc):
            fdj = plsc.load_gather(fd_v, [zi + j])
            tj = sc0(plsc.load_gather(topix_v, [zi + j]))
            half = (tj % 2) * 64
            return tuple(
                wc[c] + fdj * usel_v[j, pl.ds(half + 16 * c, 16)]
                for c in range(4))
        w = lax.fori_loop(0, _MAXR, wacc_body, (zf, zf, zf, zf))

        dth = jnp.sum(fd0 * thsel_v[pl.ds(0, 16)]
                      + fd1 * thsel_v[pl.ds(16, 16)])
        vw = jnp.float32(0.0)
        ww = jnp.float32(0.0)
        for c in range(4):
            vv = vrow_v[pl.ds(16 * c, 16)]
            vw = vw + jnp.sum(vv * w[c])
            ww = ww + jnp.sum(w[c] * w[c])
        dscore = dth + vw + 0.5 * ww
        sum_s = jnp.sum(f0 * s0 + f1 * s1)
        a = dscore - 2.0 * sum_s

        corr = (f0 * (jnp.exp(-s0 - mrow) - jnp.exp(s0 - mrow))
                + f1 * (jnp.exp(-s1 - mrow) - jnp.exp(s1 - mrow)))
        se_y = serow + jnp.sum(corr)
        av = jnp.clip(zf + a, -60.0, 60.0)
        lhs = jnp.exp(av) * serow
        rhs = uacc * jnp.maximum(se_y, 1e-30)
        acc = lhs > rhs

        nb0 = jnp.where(jnp.logical_and(acc, f0 > 0.0), 1.0 - x0, x0)
        nb1 = jnp.where(jnp.logical_and(acc, f1 > 0.0), 1.0 - x1, x1)

        plsc.store_scatter(xrow_v, [tix0], nb0, mask=lanes >= 0)
        plsc.store_scatter(xrow_v, [tix1], nb1, mask=lanes < 15)
        pltpu.sync_copy(xrow_v, out_hbm.at[r])

        @pl.when(rr < _RPW - 1)
        def _():
            pltpu.async_copy(x_hbm.at[r + 1], xrow_v, sem2)
        return carry
    lax.fori_loop(0, _RPW, row_body, 0)


def _sc_call(x, pblk, bm, m, se, radius, u_accept, v, theta, U, gflat):
    f32 = jnp.float32
    i32 = jnp.int32
    mesh = plsc.VectorSubcoreMesh(core_axis_name="c", subcore_axis_name="s")
    kern = functools.partial(
        pl.kernel,
        mesh=mesh,
        out_type=jax.ShapeDtypeStruct((_B, _D), f32),
        scratch_types=[
            pltpu.VMEM((_D,), f32),
            pltpu.VMEM((_ND, _NBW), f32),
            pltpu.VMEM((2 * 16,), i32),
            pltpu.VMEM((2 * 16,), i32),
            pltpu.VMEM((_MAXR + 1, _W), f32),
            pltpu.VMEM((_POOL,), f32),
            pltpu.VMEM((_POOL,), i32),
            pltpu.VMEM((2 * 16,), i32),
            pltpu.VMEM((2 * 16,), f32),
            pltpu.VMEM((2 * 16,), i32),
            pltpu.VMEM((2 * 16,), i32),
            pltpu.VMEM((2 * 16,), f32),
            pltpu.VMEM((2 * 16,), f32),
            pltpu.VMEM((_MAXR + 1, 2 * _K), f32),
            pltpu.VMEM((_K,), f32),
            pltpu.VMEM((2 * 16,), f32),
            pltpu.VMEM((_B,), i32),
            pltpu.VMEM((_B,), f32),
            pltpu.VMEM((_B,), f32),
            pltpu.VMEM((_B,), f32),
            pltpu.SemaphoreType.DMA,
            pltpu.SemaphoreType.DMA,
        ],
        compiler_params=pltpu.CompilerParams(needs_layout_passes=False),
    )(_sc_body)
    return kern(x, pblk, bm, m, se, radius, u_accept, v, theta, U, gflat)


def kernel(x, theta, U, radius, gumbel, u_accept):
    f32 = jnp.float32
    x = x.astype(f32)
    th2 = theta.reshape(1, _D)

    v = pl.pallas_call(
        _acc_kernel,
        grid=(_ND,),
        in_specs=[
            pl.BlockSpec((_B, _DT), lambda j: (0, j)),
            pl.BlockSpec((_DT, _K), lambda j: (j, 0)),
        ],
        out_specs=pl.BlockSpec((_B, _K), lambda j: (0, 0)),
        out_shape=jax.ShapeDtypeStruct((_B, _K), f32),
        interpret=_INTERPRET,
    )(x, U)

    outs = pl.pallas_call(
        _score_kernel,
        grid=(_ND, _NR),
        in_specs=[
            pl.BlockSpec((_RB, _DT), lambda j, i: (i, j)),
            pl.BlockSpec((_RB, _DT), lambda j, i: (i, j)),
            pl.BlockSpec((_DT, _K), lambda j, i: (j, 0)),
            pl.BlockSpec((1, _DT), lambda j, i: (0, j)),
            pl.BlockSpec((_RB, _K), lambda j, i: (i, 0)),
        ],
        out_specs=[
            pl.BlockSpec((_RB, _DT), lambda j, i: (i, j)),
            pl.BlockSpec((1, _RB, _NBW), lambda j, i: (j, i, 0)),
            pl.BlockSpec((_RB, 1), lambda j, i: (i, 0)),
            pl.BlockSpec((_RB, 1), lambda j, i: (i, 0)),
        ],
        out_shape=[
            jax.ShapeDtypeStruct((_B, _D), f32),
            jax.ShapeDtypeStruct((_ND, _B, _NBW), f32),
            jax.ShapeDtypeStruct((_B, 1), f32),
            jax.ShapeDtypeStruct((_B, 1), f32),
        ],
        scratch_shapes=[
            pltpu.VMEM((_B, 1), f32),
            pltpu.VMEM((_B, 1), f32),
        ],
        interpret=_INTERPRET,
    )(x, gumbel, U, th2, v)
    pert, bm3, m, se = outs

    return _sc_call(
        x, pert.reshape(_B * _NBLK, _W), bm3,
        m.reshape(_B), se.reshape(_B), radius.reshape(_B).astype(jnp.int32),
        u_accept, v, theta, U.reshape(_D // 2, 2 * _K),
        gumbel.reshape(_B * _D))


# speedup vs baseline: 7.6328x; 1.0428x over previous
import functools

import jax
import jax.numpy as jnp
from jax import lax
from jax.experimental import pallas as pl
from jax.experimental.pallas import tpu as pltpu
from jax.experimental.pallas import tpu_sc as plsc

_B, _D, _K = 128, 32768, 64
_DT = 4096
_ND = _D // _DT
_RB = 128
_NR = _B // _RB
_W = 128
_NBW = _DT // _W
_NBLK = _D // _W
_MAXR = 31
_NW = 32
_RPW = _B // _NW
_POOL = _MAXR * _W + 16
_NEG = -3.0e38

_INTERPRET = False


def _acc_kernel(x_ref, u_ref, v_ref):
    @pl.when(pl.program_id(0) == 0)
    def _():
        v_ref[...] = jnp.zeros_like(v_ref)
    v_ref[...] += jnp.dot(x_ref[...], u_ref[...],
                          preferred_element_type=jnp.float32)


def _score_kernel(x_ref, g_ref, u_ref, th_ref, v_ref,
                  pert_ref, bm_ref, m_ref, se_ref,
                  macc, seacc):
    j = pl.program_id(0)
    x = x_ref[...]
    grad = th_ref[...] + lax.dot_general(
        v_ref[...], u_ref[...], (((1,), (1,)), ((), ())),
        preferred_element_type=jnp.float32)
    s = (0.5 - x) * grad
    pert = s + g_ref[...]
    pert_ref[...] = pert
    tm = jnp.max(s, axis=-1, keepdims=True)
    tse = jnp.sum(jnp.exp(s - tm), axis=-1, keepdims=True)
    row0 = pl.program_id(1) * _RB
    bm_ref[...] = jnp.max(pert.reshape(_RB, _NBW, _W), axis=-1)[None]

    @pl.when(j == 0)
    def _():
        macc[pl.ds(row0, _RB), :] = tm
        seacc[pl.ds(row0, _RB), :] = tse

    @pl.when(j > 0)
    def _():
        m_old = macc[pl.ds(row0, _RB), :]
        se_old = seacc[pl.ds(row0, _RB), :]
        m_new = jnp.maximum(m_old, tm)
        macc[pl.ds(row0, _RB), :] = m_new
        seacc[pl.ds(row0, _RB), :] = (se_old * jnp.exp(m_old - m_new)
                                      + tse * jnp.exp(tm - m_new))

    @pl.when(j == _ND - 1)
    def _():
        m_ref[...] = macc[pl.ds(row0, _RB), :]
        se_ref[...] = seacc[pl.ds(row0, _RB), :]


def _sc_body(x_hbm, pblk_hbm, bm_hbm, m_hbm, se_hbm, rad_hbm, uac_hbm,
             v_hbm, th_hbm, utab_hbm, gf_hbm,
             out_hbm,
             xrow_v, bm_v, blkid_v, cbase_v, blocks_v, pool_v, poolix_v,
             topix_v, topval_v, gidx_v, uidx_v, gsel_v, thsel_v, usel_v,
             vrow_v, fd_v, rad_v, uac_v, m_v, se_v, sem, sem2):
    i32 = jnp.int32
    f32 = jnp.float32
    lanes = lax.iota(i32, 16)
    zi = jnp.zeros((16,), i32)
    zf = jnp.zeros((16,), f32)

    def sc0(vec):
        return lax.squeeze(lax.slice(vec, (0,), (1,)), (0,))

    wid = lax.axis_index("s") * 2 + lax.axis_index("c")
    base = wid * _RPW

    pltpu.sync_copy(rad_hbm, rad_v)
    pltpu.sync_copy(uac_hbm, uac_v)
    pltpu.sync_copy(m_hbm, m_v)
    pltpu.sync_copy(se_hbm, se_v)

    pltpu.async_copy(x_hbm.at[base], xrow_v, sem2)

    def row_body(rr, carry):
        r = base + rr
        pltpu.sync_copy(bm_hbm.at[:, r], bm_v)
        pltpu.sync_copy(v_hbm.at[r], vrow_v)

        blkid_v[pl.ds(0, 16)] = zi
        blkid_v[pl.ds(16, 16)] = zi
        cbase_v[pl.ds(0, 16)] = zi
        cbase_v[pl.ds(16, 16)] = zi

        def ext_blk(j, _t):
            rv = jnp.full((16,), _NEG, f32)
            for k in range(16):
                rv = jnp.maximum(
                    rv, bm_v[k // 2, pl.ds((k % 2) * 16, 16)])
            gmax = jnp.max(rv)
            posv = jnp.full((16,), -1, i32)
            for k in range(16):
                pv = bm_v[k // 2, pl.ds((k % 2) * 16, 16)]
                posv = jnp.maximum(
                    posv, jnp.where(pv == gmax, lanes + 16 * k, -1))
            pos = jnp.max(posv)
            plsc.store_scatter(bm_v, [zi + pos // _NBW, zi + pos % _NBW],
                               zf + _NEG, mask=lanes == 0)
            plsc.store_scatter(blkid_v, [zi + j], zi + (r * _NBLK + pos),
                               mask=lanes == 0)
            plsc.store_scatter(cbase_v, [zi + j], zi + pos * _W,
                               mask=lanes == 0)
            return gmax
        t31 = lax.fori_loop(0, _MAXR, ext_blk, _NEG)

        pltpu.async_copy(pblk_hbm.at[blkid_v], blocks_v, sem).wait()

        t31v = zf + t31

        def comp_blk(jj, cur):
            cb = plsc.load_gather(cbase_v, [zi + jj])
            for k in range(8):
                pv = blocks_v[jj, pl.ds(16 * k, 16)]
                msk = pv >= t31v
                plsc.store_compressed(pool_v.at[pl.ds(cur, 16)], pv, mask=msk)
                plsc.store_compressed(poolix_v.at[pl.ds(cur, 16)],
                                      cb + (16 * k) + lanes, mask=msk)
                cur = cur + sc0(plsc.all_reduce_population_count(msk))
            return cur
        psize = lax.fori_loop(0, _MAXR, comp_blk, jnp.int32(0))
        pool_v[pl.ds(psize, 16)] = zf + _NEG
        poolix_v[pl.ds(psize, 16)] = zi

        nv = (psize + 15) // 16
        topix_v[pl.ds(0, 16)] = zi
        topix_v[pl.ds(16, 16)] = zi

        def ext_pool(j, _u):
            def scan1(k, rv):
                return jnp.maximum(rv, pool_v[pl.ds(16 * k, 16)])
            rv = lax.fori_loop(0, nv, scan1, jnp.full((16,), _NEG, f32))
            gmax = jnp.max(rv)

            def scan2(k, c):
                posv, idxv = c
                pv = pool_v[pl.ds(16 * k, 16)]
                hit = pv == gmax
                posv = jnp.maximum(
                    posv, jnp.where(hit, lanes + 16 * k, -1))
                idxv = jnp.maximum(
                    idxv, jnp.where(hit, poolix_v[pl.ds(16 * k, 16)], -1))
                return posv, idxv
            posv, idxv = lax.fori_loop(
                0, nv, scan2,
                (jnp.full((16,), -1, i32), jnp.full((16,), -1, i32)))
            pos = jnp.max(posv)
            col = jnp.max(idxv)
            plsc.store_scatter(pool_v, [zi + pos], zf + _NEG,
                               mask=lanes == 0)
            plsc.store_scatter(topix_v, [zi + j], zi + col,
                               mask=lanes == 0)
            plsc.store_scatter(topval_v, [zi + j], zf + gmax,
                               mask=lanes == 0)
            return 0
        lax.fori_loop(0, _MAXR, ext_pool, 0)

        tix0 = topix_v[pl.ds(0, 16)]
        tix1 = topix_v[pl.ds(16, 16)]
        gidx_v[pl.ds(0, 16)] = tix0 + r * _D
        gidx_v[pl.ds(16, 16)] = tix1 + r * _D
        uidx_v[pl.ds(0, 16)] = tix0 // 2
        uidx_v[pl.ds(16, 16)] = tix1 // 2
        cg = pltpu.async_copy(gf_hbm.at[gidx_v], gsel_v, sem)
        ct = pltpu.async_copy(th_hbm.at[topix_v], thsel_v, sem)
        cu = pltpu.async_copy(utab_hbm.at[uidx_v], usel_v, sem)
        pltpu.make_async_copy(x_hbm.at[r], xrow_v, sem2).wait()
        x0 = plsc.load_gather(xrow_v, [tix0])
        x1 = plsc.load_gather(xrow_v, [tix1])
        cg.wait()
        ct.wait()
        cu.wait()

        rad = plsc.load_gather(rad_v, [zi + r])
        uacc = plsc.load_gather(uac_v, [zi + r])
        mrow = plsc.load_gather(m_v, [zi + r])
        serow = plsc.load_gather(se_v, [zi + r])

        tv0 = topval_v[pl.ds(0, 16)]
        tv1 = topval_v[pl.ds(16, 16)]
        s0 = tv0 - gsel_v[pl.ds(0, 16)]
        s1 = tv1 - gsel_v[pl.ds(16, 16)]
        d0 = 1.0 - 2.0 * x0
        d1 = 1.0 - 2.0 * x1
        f0 = jnp.where(lanes < rad, 1.0, 0.0)
        f1 = jnp.where(lanes + 16 < rad, 1.0, 0.0)
        fd0 = f0 * d0
        fd1 = f1 * d1
        fd_v[pl.ds(0, 16)] = fd0
        fd_v[pl.ds(16, 16)] = fd1

        def wacc_body(j, wc):
            fdj = plsc.load_gather(fd_v, [zi + j])
            tj = sc0(plsc.load_gather(topix_v, [zi + j]))
            half = (tj % 2) * 64
            return tuple(
                wc[c] + fdj * usel_v[j, pl.ds(half + 16 * c, 16)]
                for c in range(4))
        w = lax.fori_loop(0, _MAXR, wacc_body, (zf, zf, zf, zf))

        dth = jnp.sum(fd0 * thsel_v[pl.ds(0, 16)]
                      + fd1 * thsel_v[pl.ds(16, 16)])
        vw = jnp.float32(0.0)
        ww = jnp.float32(0.0)
        for c in range(4):
            vv = vrow_v[pl.ds(16 * c, 16)]
            vw = vw + jnp.sum(vv * w[c])
            ww = ww + jnp.sum(w[c] * w[c])
        dscore = dth + vw + 0.5 * ww
        sum_s = jnp.sum(f0 * s0 + f1 * s1)
        a = dscore - 2.0 * sum_s

        corr = (f0 * (jnp.exp(-s0 - mrow) - jnp.exp(s0 - mrow))
                + f1 * (jnp.exp(-s1 - mrow) - jnp.exp(s1 - mrow)))
        se_y = serow + jnp.sum(corr)
        av = jnp.clip(zf + a, -60.0, 60.0)
        lhs = jnp.exp(av) * serow
        rhs = uacc * jnp.maximum(se_y, 1e-30)
        acc = lhs > rhs

        nb0 = jnp.where(jnp.logical_and(acc, f0 > 0.0), 1.0 - x0, x0)
        nb1 = jnp.where(jnp.logical_and(acc, f1 > 0.0), 1.0 - x1, x1)

        plsc.store_scatter(xrow_v, [tix0], nb0, mask=lanes >= 0)
        plsc.store_scatter(xrow_v, [tix1], nb1, mask=lanes < 15)
        pltpu.sync_copy(xrow_v, out_hbm.at[r])

        @pl.when(rr < _RPW - 1)
        def _():
            pltpu.async_copy(x_hbm.at[r + 1], xrow_v, sem2)
        return carry
    lax.fori_loop(0, _RPW, row_body, 0)


def _sc_call(x, pblk, bm, m, se, radius, u_accept, v, theta, U, gflat):
    f32 = jnp.float32
    i32 = jnp.int32
    mesh = plsc.VectorSubcoreMesh(core_axis_name="c", subcore_axis_name="s")
    kern = functools.partial(
        pl.kernel,
        mesh=mesh,
        out_type=jax.ShapeDtypeStruct((_B, _D), f32),
        scratch_types=[
            pltpu.VMEM((_D,), f32),
            pltpu.VMEM((_ND, _NBW), f32),
            pltpu.VMEM((2 * 16,), i32),
            pltpu.VMEM((2 * 16,), i32),
            pltpu.VMEM((_MAXR + 1, _W), f32),
            pltpu.VMEM((_POOL,), f32),
            pltpu.VMEM((_POOL,), i32),
            pltpu.VMEM((2 * 16,), i32),
            pltpu.VMEM((2 * 16,), f32),
            pltpu.VMEM((2 * 16,), i32),
            pltpu.VMEM((2 * 16,), i32),
            pltpu.VMEM((2 * 16,), f32),
            pltpu.VMEM((2 * 16,), f32),
            pltpu.VMEM((_MAXR + 1, 2 * _K), f32),
            pltpu.VMEM((_K,), f32),
            pltpu.VMEM((2 * 16,), f32),
            pltpu.VMEM((_B,), i32),
            pltpu.VMEM((_B,), f32),
            pltpu.VMEM((_B,), f32),
            pltpu.VMEM((_B,), f32),
            pltpu.SemaphoreType.DMA,
            pltpu.SemaphoreType.DMA,
        ],
        compiler_params=pltpu.CompilerParams(needs_layout_passes=False),
    )(_sc_body)
    return kern(x, pblk, bm, m, se, radius, u_accept, v, theta, U, gflat)


def kernel(x, theta, U, radius, gumbel, u_accept):
    f32 = jnp.float32
    x = x.astype(f32)
    th2 = theta.reshape(1, _D)

    v = pl.pallas_call(
        _acc_kernel,
        grid=(_ND,),
        in_specs=[
            pl.BlockSpec((_B, _DT), lambda j: (0, j)),
            pl.BlockSpec((_DT, _K), lambda j: (j, 0)),
        ],
        out_specs=pl.BlockSpec((_B, _K), lambda j: (0, 0)),
        out_shape=jax.ShapeDtypeStruct((_B, _K), f32),
        interpret=_INTERPRET,
    )(x, U)

    outs = pl.pallas_call(
        _score_kernel,
        grid=(_ND, _NR),
        in_specs=[
            pl.BlockSpec((_RB, _DT), lambda j, i: (i, j)),
            pl.BlockSpec((_RB, _DT), lambda j, i: (i, j)),
            pl.BlockSpec((_DT, _K), lambda j, i: (j, 0)),
            pl.BlockSpec((1, _DT), lambda j, i: (0, j)),
            pl.BlockSpec((_RB, _K), lambda j, i: (i, 0)),
        ],
        out_specs=[
            pl.BlockSpec((_RB, _DT), lambda j, i: (i, j)),
            pl.BlockSpec((1, _RB, _NBW), lambda j, i: (j, i, 0)),
            pl.BlockSpec((_RB, 1), lambda j, i: (i, 0)),
            pl.BlockSpec((_RB, 1), lambda j, i: (i, 0)),
        ],
        out_shape=[
            jax.ShapeDtypeStruct((_B, _D), f32),
            jax.ShapeDtypeStruct((_ND, _B, _NBW), f32),
            jax.ShapeDtypeStruct((_B, 1), f32),
            jax.ShapeDtypeStruct((_B, 1), f32),
        ],
        scratch_shapes=[
            pltpu.VMEM((_B, 1), f32),
            pltpu.VMEM((_B, 1), f32),
        ],
        interpret=_INTERPRET,
    )(x, gumbel, U, th2, v)
    pert, bm3, m, se = outs

    return _sc_call(
        x, pert.reshape(_B * _NBLK, _W), bm3,
        m.reshape(_B), se.reshape(_B), radius.reshape(_B).astype(jnp.int32),
        u_accept, v, theta, U.reshape(_D // 2, 2 * _K),
        gumbel.reshape(_B * _D))


# speedup vs baseline: 7.6605x; 1.0036x over previous
import functools

import jax
import jax.numpy as jnp
from jax import lax
from jax.experimental import pallas as pl
from jax.experimental.pallas import tpu as pltpu
from jax.experimental.pallas import tpu_sc as plsc

_B, _D, _K = 128, 32768, 64
_DT = 4096
_ND = _D // _DT
_RB = 128
_NR = _B // _RB
_W = 128
_NBW = _DT // _W
_NBLK = _D // _W
_MAXR = 31
_NW = 32
_RPW = _B // _NW
_POOL = _MAXR * _W + 16
_NEG = -3.0e38


def _acc_kernel(x_ref, u_ref, v_ref):
    @pl.when(pl.program_id(0) == 0)
    def _():
        v_ref[...] = jnp.zeros_like(v_ref)
    v_ref[...] += jnp.dot(x_ref[...], u_ref[...],
                          preferred_element_type=jnp.float32)


def _score_kernel(x_ref, g_ref, u_ref, th_ref, v_ref,
                  pert_ref, bm_ref, m_ref, se_ref,
                  macc, seacc):
    j = pl.program_id(0)
    x = x_ref[...]
    grad = th_ref[...] + lax.dot_general(
        v_ref[...], u_ref[...], (((1,), (1,)), ((), ())),
        preferred_element_type=jnp.float32)
    s = (0.5 - x) * grad
    pert = s + g_ref[...]
    pert_ref[...] = pert
    tm = jnp.max(s, axis=-1, keepdims=True)
    tse = jnp.sum(jnp.exp(s - tm), axis=-1, keepdims=True)
    row0 = pl.program_id(1) * _RB
    bm_ref[...] = jnp.max(pert.reshape(_RB, _NBW, _W), axis=-1)[None]

    @pl.when(j == 0)
    def _():
        macc[pl.ds(row0, _RB), :] = tm
        seacc[pl.ds(row0, _RB), :] = tse

    @pl.when(j > 0)
    def _():
        m_old = macc[pl.ds(row0, _RB), :]
        se_old = seacc[pl.ds(row0, _RB), :]
        m_new = jnp.maximum(m_old, tm)
        macc[pl.ds(row0, _RB), :] = m_new
        seacc[pl.ds(row0, _RB), :] = (se_old * jnp.exp(m_old - m_new)
                                      + tse * jnp.exp(tm - m_new))

    @pl.when(j == _ND - 1)
    def _():
        m_ref[...] = macc[pl.ds(row0, _RB), :]
        se_ref[...] = seacc[pl.ds(row0, _RB), :]


def _sc_body(x_hbm, pblk_hbm, bm_hbm, m_hbm, se_hbm, rad_hbm, uac_hbm,
             v_hbm, th_hbm, utab_hbm, gf_hbm,
             out_hbm,
             xrow_v, bm_v, blkid_v, cbase_v, blocks_v, pool_v, poolix_v,
             topix_v, topval_v, gidx_v, uidx_v, gsel_v, thsel_v, usel_v,
             vrow_v, fd_v, rad_v, uac_v, m_v, se_v, sem, sem2):
    i32 = jnp.int32
    f32 = jnp.float32
    lanes = lax.iota(i32, 16)
    zi = jnp.zeros((16,), i32)
    zf = jnp.zeros((16,), f32)

    def sc0(vec):
        return lax.squeeze(lax.slice(vec, (0,), (1,)), (0,))

    wid = lax.axis_index("s") * 2 + lax.axis_index("c")
    base = wid * _RPW

    pltpu.sync_copy(rad_hbm, rad_v)
    pltpu.sync_copy(uac_hbm, uac_v)
    pltpu.sync_copy(m_hbm, m_v)
    pltpu.sync_copy(se_hbm, se_v)

    pltpu.async_copy(x_hbm.at[base], xrow_v, sem2)

    def row_body(rr, carry):
        r = base + rr
        pltpu.sync_copy(bm_hbm.at[:, r], bm_v)
        pltpu.sync_copy(v_hbm.at[r], vrow_v)

        blkid_v[pl.ds(0, 16)] = zi
        blkid_v[pl.ds(16, 16)] = zi
        cbase_v[pl.ds(0, 16)] = zi
        cbase_v[pl.ds(16, 16)] = zi

        def ext_blk(j, _t):
            rv = jnp.full((16,), _NEG, f32)
            for k in range(16):
                rv = jnp.maximum(
                    rv, bm_v[k // 2, pl.ds((k % 2) * 16, 16)])
            gmax = jnp.max(rv)
            posv = jnp.full((16,), -1, i32)
            for k in range(16):
                pv = bm_v[k // 2, pl.ds((k % 2) * 16, 16)]
                posv = jnp.maximum(
                    posv, jnp.where(pv == gmax, lanes + 16 * k, -1))
            pos = jnp.max(posv)
            plsc.store_scatter(bm_v, [zi + pos // _NBW, zi + pos % _NBW],
                               zf + _NEG, mask=lanes == 0)
            plsc.store_scatter(blkid_v, [zi + j], zi + (r * _NBLK + pos),
                               mask=lanes == 0)
            plsc.store_scatter(cbase_v, [zi + j], zi + pos * _W,
                               mask=lanes == 0)
            return gmax
        t31 = lax.fori_loop(0, _MAXR, ext_blk, _NEG)

        pltpu.async_copy(pblk_hbm.at[blkid_v], blocks_v, sem).wait()

        t31v = zf + t31

        def comp_blk(jj, cur):
            cb = plsc.load_gather(cbase_v, [zi + jj])
            for k in range(8):
                pv = blocks_v[jj, pl.ds(16 * k, 16)]
                msk = pv >= t31v
                plsc.store_compressed(pool_v.at[pl.ds(cur, 16)], pv, mask=msk)
                plsc.store_compressed(poolix_v.at[pl.ds(cur, 16)],
                                      cb + (16 * k) + lanes, mask=msk)
                cur = cur + sc0(plsc.all_reduce_population_count(msk))
            return cur
        psize = lax.fori_loop(0, _MAXR, comp_blk, jnp.int32(0))
        pool_v[pl.ds(psize, 16)] = zf + _NEG
        poolix_v[pl.ds(psize, 16)] = zi

        nv = (psize + 15) // 16
        topix_v[pl.ds(0, 16)] = zi
        topix_v[pl.ds(16, 16)] = zi

        def ext_pool(j, _u):
            def scan1(k, rv):
                return jnp.maximum(rv, pool_v[pl.ds(16 * k, 16)])
            rv = lax.fori_loop(0, nv, scan1, jnp.full((16,), _NEG, f32))
            gmax = jnp.max(rv)

            def scan2(k, c):
                posv, idxv = c
                pv = pool_v[pl.ds(16 * k, 16)]
                hit = pv == gmax
                posv = jnp.maximum(
                    posv, jnp.where(hit, lanes + 16 * k, -1))
                idxv = jnp.maximum(
                    idxv, jnp.where(hit, poolix_v[pl.ds(16 * k, 16)], -1))
                return posv, idxv
            posv, idxv = lax.fori_loop(
                0, nv, scan2,
                (jnp.full((16,), -1, i32), jnp.full((16,), -1, i32)))
            pos = jnp.max(posv)
            col = jnp.max(idxv)
            plsc.store_scatter(pool_v, [zi + pos], zf + _NEG,
                               mask=lanes == 0)
            plsc.store_scatter(topix_v, [zi + j], zi + col,
                               mask=lanes == 0)
            plsc.store_scatter(topval_v, [zi + j], zf + gmax,
                               mask=lanes == 0)
            return 0
        lax.fori_loop(0, _MAXR, ext_pool, 0)

        tix0 = topix_v[pl.ds(0, 16)]
        tix1 = topix_v[pl.ds(16, 16)]
        gidx_v[pl.ds(0, 16)] = tix0 + r * _D
        gidx_v[pl.ds(16, 16)] = tix1 + r * _D
        uidx_v[pl.ds(0, 16)] = tix0 // 2
        uidx_v[pl.ds(16, 16)] = tix1 // 2
        cg = pltpu.async_copy(gf_hbm.at[gidx_v], gsel_v, sem)
        ct = pltpu.async_copy(th_hbm.at[topix_v], thsel_v, sem)
        cu = pltpu.async_copy(utab_hbm.at[uidx_v], usel_v, sem)
        pltpu.make_async_copy(x_hbm.at[r], xrow_v, sem2).wait()
        x0 = plsc.load_gather(xrow_v, [tix0])
        x1 = plsc.load_gather(xrow_v, [tix1])
        cg.wait()
        ct.wait()
        cu.wait()

        rad = plsc.load_gather(rad_v, [zi + r])
        uacc = plsc.load_gather(uac_v, [zi + r])
        mrow = plsc.load_gather(m_v, [zi + r])
        serow = plsc.load_gather(se_v, [zi + r])

        tv0 = topval_v[pl.ds(0, 16)]
        tv1 = topval_v[pl.ds(16, 16)]
        s0 = tv0 - gsel_v[pl.ds(0, 16)]
        s1 = tv1 - gsel_v[pl.ds(16, 16)]
        d0 = 1.0 - 2.0 * x0
        d1 = 1.0 - 2.0 * x1
        f0 = jnp.where(lanes < rad, 1.0, 0.0)
        f1 = jnp.where(lanes + 16 < rad, 1.0, 0.0)
        fd0 = f0 * d0
        fd1 = f1 * d1
        fd_v[pl.ds(0, 16)] = fd0
        fd_v[pl.ds(16, 16)] = fd1

        def wacc_body(j, wc):
            fdj = plsc.load_gather(fd_v, [zi + j])
            tj = sc0(plsc.load_gather(topix_v, [zi + j]))
            half = (tj % 2) * 64
            return tuple(
                wc[c] + fdj * usel_v[j, pl.ds(half + 16 * c, 16)]
                for c in range(4))
        w = lax.fori_loop(0, _MAXR, wacc_body, (zf, zf, zf, zf))

        dth = jnp.sum(fd0 * thsel_v[pl.ds(0, 16)]
                      + fd1 * thsel_v[pl.ds(16, 16)])
        vw = jnp.float32(0.0)
        ww = jnp.float32(0.0)
        for c in range(4):
            vv = vrow_v[pl.ds(16 * c, 16)]
            vw = vw + jnp.sum(vv * w[c])
            ww = ww + jnp.sum(w[c] * w[c])
        dscore = dth + vw + 0.5 * ww
        sum_s = jnp.sum(f0 * s0 + f1 * s1)
        a = dscore - 2.0 * sum_s

        corr = (f0 * (jnp.exp(-s0 - mrow) - jnp.exp(s0 - mrow))
                + f1 * (jnp.exp(-s1 - mrow) - jnp.exp(s1 - mrow)))
        se_y = serow + jnp.sum(corr)
        av = jnp.clip(zf + a, -60.0, 60.0)
        lhs = jnp.exp(av) * serow
        rhs = uacc * jnp.maximum(se_y, 1e-30)
        acc = lhs > rhs

        nb0 = jnp.where(jnp.logical_and(acc, f0 > 0.0), 1.0 - x0, x0)
        nb1 = jnp.where(jnp.logical_and(acc, f1 > 0.0), 1.0 - x1, x1)

        plsc.store_scatter(xrow_v, [tix0], nb0, mask=lanes >= 0)
        plsc.store_scatter(xrow_v, [tix1], nb1, mask=lanes < 15)
        pltpu.sync_copy(xrow_v, out_hbm.at[r])

        @pl.when(rr < _RPW - 1)
        def _():
            pltpu.async_copy(x_hbm.at[r + 1], xrow_v, sem2)
        return carry
    lax.fori_loop(0, _RPW, row_body, 0)


def _sc_call(x, pblk, bm, m, se, radius, u_accept, v, theta, U, gflat):
    f32 = jnp.float32
    i32 = jnp.int32
    mesh = plsc.VectorSubcoreMesh(core_axis_name="c", subcore_axis_name="s")
    kern = functools.partial(
        pl.kernel,
        mesh=mesh,
        out_type=jax.ShapeDtypeStruct((_B, _D), f32),
        scratch_types=[
            pltpu.VMEM((_D,), f32),
            pltpu.VMEM((_ND, _NBW), f32),
            pltpu.VMEM((2 * 16,), i32),
            pltpu.VMEM((2 * 16,), i32),
            pltpu.VMEM((_MAXR + 1, _W), f32),
            pltpu.VMEM((_POOL,), f32),
            pltpu.VMEM((_POOL,), i32),
            pltpu.VMEM((2 * 16,), i32),
            pltpu.VMEM((2 * 16,), f32),
            pltpu.VMEM((2 * 16,), i32),
            pltpu.VMEM((2 * 16,), i32),
            pltpu.VMEM((2 * 16,), f32),
            pltpu.VMEM((2 * 16,), f32),
            pltpu.VMEM((_MAXR + 1, 2 * _K), f32),
            pltpu.VMEM((_K,), f32),
            pltpu.VMEM((2 * 16,), f32),
            pltpu.VMEM((_B,), i32),
            pltpu.VMEM((_B,), f32),
            pltpu.VMEM((_B,), f32),
            pltpu.VMEM((_B,), f32),
            pltpu.SemaphoreType.DMA,
            pltpu.SemaphoreType.DMA,
        ],
        compiler_params=pltpu.CompilerParams(needs_layout_passes=False),
    )(_sc_body)
    return kern(x, pblk, bm, m, se, radius, u_accept, v, theta, U, gflat)


def kernel(x, theta, U, radius, gumbel, u_accept):
    f32 = jnp.float32
    x = x.astype(f32)
    th2 = theta.reshape(1, _D)

    v = pl.pallas_call(
        _acc_kernel,
        grid=(_ND,),
        in_specs=[
            pl.BlockSpec((_B, _DT), lambda j: (0, j)),
            pl.BlockSpec((_DT, _K), lambda j: (j, 0)),
        ],
        out_specs=pl.BlockSpec((_B, _K), lambda j: (0, 0)),
        out_shape=jax.ShapeDtypeStruct((_B, _K), f32),
    )(x, U)

    outs = pl.pallas_call(
        _score_kernel,
        grid=(_ND, _NR),
        in_specs=[
            pl.BlockSpec((_RB, _DT), lambda j, i: (i, j)),
            pl.BlockSpec((_RB, _DT), lambda j, i: (i, j)),
            pl.BlockSpec((_DT, _K), lambda j, i: (j, 0)),
            pl.BlockSpec((1, _DT), lambda j, i: (0, j)),
            pl.BlockSpec((_RB, _K), lambda j, i: (i, 0)),
        ],
        out_specs=[
            pl.BlockSpec((_RB, _DT), lambda j, i: (i, j)),
            pl.BlockSpec((1, _RB, _NBW), lambda j, i: (j, i, 0)),
            pl.BlockSpec((_RB, 1), lambda j, i: (i, 0)),
            pl.BlockSpec((_RB, 1), lambda j, i: (i, 0)),
        ],
        out_shape=[
            jax.ShapeDtypeStruct((_B, _D), f32),
            jax.ShapeDtypeStruct((_ND, _B, _NBW), f32),
            jax.ShapeDtypeStruct((_B, 1), f32),
            jax.ShapeDtypeStruct((_B, 1), f32),
        ],
        scratch_shapes=[
            pltpu.VMEM((_B, 1), f32),
            pltpu.VMEM((_B, 1), f32),
        ],
    )(x, gumbel, U, th2, v)
    pert, bm3, m, se = outs

    return _sc_call(
        x, pert.reshape(_B * _NBLK, _W), bm3,
        m.reshape(_B), se.reshape(_B), radius.reshape(_B).astype(jnp.int32),
        u_accept, v, theta, U.reshape(_D // 2, 2 * _K),
        gumbel.reshape(_B * _D))
